# baseline (speedup 1.0000x reference)
import functools

import jax
import jax.numpy as jnp
from jax import lax
from jax.experimental import pallas as pl
from jax.experimental.pallas import tpu as pltpu
from jax.experimental.pallas import tpu_sc as plsc

_B = 4
_S = 2048
_D = 4096
_L = 16


def _sc_body(hidden_hbm, ids_hbm, out_hbm, ids_v):
    nc = 2
    wid = lax.axis_index("s") * nc + lax.axis_index("c")

    @pl.when(wid < _B)
    def _():
        b = wid
        pltpu.sync_copy(ids_hbm.at[b], ids_v)

        lane = lax.iota(jnp.int32, _L)
        acc = jnp.full((_L,), -1, jnp.int32)
        for i in range(_S // _L):
            vals = ids_v[pl.ds(i * _L, _L)]
            key = vals * _S + (_S - 1 - i * _L - lane)
            acc = jnp.maximum(acc, key)
        best = acc[0]
        for j in range(1, _L):
            best = jnp.maximum(best, acc[j])
        idx = (_S - 1) - (best & (_S - 1))

        pltpu.sync_copy(hidden_hbm.at[b * _S + idx], out_hbm.at[b])


@jax.jit
def kernel(last_hidden_state, input_ids):
    hidden2d = last_hidden_state.reshape(_B * _S, _D)
    run = pl.kernel(
        _sc_body,
        out_type=jax.ShapeDtypeStruct((_B, _D), jnp.float32),
        mesh=plsc.VectorSubcoreMesh(core_axis_name="c", subcore_axis_name="s"),
        scratch_types=[
            pltpu.VMEM((_S,), jnp.int32),
        ],
    )
    return run(hidden2d, input_ids)


# speedup vs baseline: 9.3064x; 9.3064x over previous
import jax
import jax.numpy as jnp
from jax import lax
from jax.experimental import pallas as pl
from jax.experimental.pallas import tpu as pltpu

_B = 4
_S = 2048
_D = 4096


def _tc_body(ids_ref, hidden_hbm, out_ref, sem):
    col = lax.broadcasted_iota(jnp.int32, (1, _S), 1)
    copies = []
    for b in range(_B):
        key = ids_ref[b : b + 1, :] * _S + ((_S - 1) - col)
        best = jnp.max(key)
        idx = (_S - 1) - (best & (_S - 1))
        copy = pltpu.make_async_copy(
            hidden_hbm.at[pl.ds(b * _S + idx, 1), :],
            out_ref.at[pl.ds(b, 1), :],
            sem,
        )
        copy.start()
        copies.append(copy)
    for copy in copies:
        copy.wait()


@jax.jit
def kernel(last_hidden_state, input_ids):
    hidden2d = last_hidden_state.reshape(_B * _S, _D)
    return pl.pallas_call(
        _tc_body,
        out_shape=jax.ShapeDtypeStruct((_B, _D), jnp.float32),
        in_specs=[
            pl.BlockSpec(memory_space=pltpu.VMEM),
            pl.BlockSpec(memory_space=pltpu.MemorySpace.HBM),
        ],
        out_specs=pl.BlockSpec(memory_space=pltpu.VMEM),
        scratch_shapes=[pltpu.SemaphoreType.DMA],
    )(input_ids, hidden2d)
